# Initial kernel scaffold; baseline (speedup 1.0000x reference)
import jax
import jax.numpy as jnp
from jax.experimental import pallas as pl


def kernel(relative_position_bias_table, relative_position_index):
    raise NotImplementedError("write your pallas kernel here")


# speedup vs baseline: 5.1315x; 5.1315x over previous
import functools

import jax
import jax.numpy as jnp
from jax import lax
from jax.experimental import pallas as pl
from jax.experimental.pallas import tpu as pltpu
from jax.experimental.pallas import tpu_sc as plsc

NUM_HEADS = 16
N_POS = 256 * 256
TABLE_WORDS = 961 * 16

_info = plsc.get_sparse_core_info()
NC, NS, L = _info.num_cores, _info.num_subcores, _info.num_lanes
NW = NC * NS
CHUNK = N_POS // NW


def _sc_body(table_hbm, idx_hbm, out_hbm, table_v, idx_v, out_v):
    wid = lax.axis_index("s") * NC + lax.axis_index("c")
    base = wid * CHUNK
    pltpu.sync_copy(table_hbm, table_v)
    pltpu.sync_copy(idx_hbm.at[pl.ds(base, CHUNK)], idx_v)

    def body(i, carry):
        c0 = i * L
        idx16 = idx_v[pl.ds(c0, L)]
        scaled = idx16 * NUM_HEADS
        for h in range(NUM_HEADS):
            vals = plsc.load_gather(table_v, [scaled + h])
            out_v[h, pl.ds(c0, L)] = vals
        return carry

    lax.fori_loop(0, CHUNK // L, body, 0)
    pltpu.sync_copy(out_v, out_hbm.at[:, pl.ds(base, CHUNK)])


_sc_gather = functools.partial(
    pl.kernel,
    mesh=plsc.VectorSubcoreMesh(core_axis_name="c", subcore_axis_name="s"),
    out_type=jax.ShapeDtypeStruct((NUM_HEADS, N_POS), jnp.float32),
    compiler_params=pltpu.CompilerParams(needs_layout_passes=False),
    scratch_types=[
        pltpu.VMEM((TABLE_WORDS,), jnp.float32),
        pltpu.VMEM((CHUNK,), jnp.int32),
        pltpu.VMEM((NUM_HEADS, CHUNK), jnp.float32),
    ],
)(_sc_body)


def kernel(relative_position_bias_table, relative_position_index):
    bs2, ws2 = relative_position_index.shape
    table_flat = relative_position_bias_table.reshape(-1)
    idx_flat = relative_position_index.reshape(-1).astype(jnp.int32)
    out = _sc_gather(table_flat, idx_flat)
    return out.reshape(1, NUM_HEADS, 1, bs2, ws2)


# speedup vs baseline: 6.4291x; 1.2529x over previous
import functools

import jax
import jax.numpy as jnp
from jax import lax
from jax.experimental import pallas as pl
from jax.experimental.pallas import tpu as pltpu
from jax.experimental.pallas import tpu_sc as plsc

NUM_HEADS = 16
N_POS = 256 * 256
TABLE_WORDS = 961 * 16

_info = plsc.get_sparse_core_info()
NC, NS, L = _info.num_cores, _info.num_subcores, _info.num_lanes
NW = NC * NS
CHUNK = N_POS // NW


def _sc_body(table_hbm, idx_hbm, out_hbm, table_v, idx_v, out_v):
    wid = lax.axis_index("s") * NC + lax.axis_index("c")
    base = wid * CHUNK
    pltpu.sync_copy(table_hbm, table_v)
    pltpu.sync_copy(idx_hbm.at[pl.ds(base, CHUNK)], idx_v)

    @plsc.parallel_loop(0, CHUNK // L, unroll=4)
    def body(i):
        c0 = i * L
        idx16 = idx_v[pl.ds(c0, L)]
        scaled = idx16 * NUM_HEADS
        for h in range(NUM_HEADS):
            vals = plsc.load_gather(table_v, [scaled + h])
            out_v[h, pl.ds(c0, L)] = vals
    pltpu.sync_copy(out_v, out_hbm.at[:, pl.ds(base, CHUNK)])


_sc_gather = functools.partial(
    pl.kernel,
    mesh=plsc.VectorSubcoreMesh(core_axis_name="c", subcore_axis_name="s"),
    out_type=jax.ShapeDtypeStruct((NUM_HEADS, N_POS), jnp.float32),
    compiler_params=pltpu.CompilerParams(needs_layout_passes=False),
    scratch_types=[
        pltpu.VMEM((TABLE_WORDS,), jnp.float32),
        pltpu.VMEM((CHUNK,), jnp.int32),
        pltpu.VMEM((NUM_HEADS, CHUNK), jnp.float32),
    ],
)(_sc_body)


def kernel(relative_position_bias_table, relative_position_index):
    bs2, ws2 = relative_position_index.shape
    table_flat = relative_position_bias_table.reshape(-1)
    idx_flat = relative_position_index.reshape(-1).astype(jnp.int32)
    out = _sc_gather(table_flat, idx_flat)
    return out.reshape(1, NUM_HEADS, 1, bs2, ws2)
